# baseline (speedup 1.0000x reference)
import functools

import jax
import jax.numpy as jnp
from jax import lax
from jax.experimental import pallas as pl
from jax.experimental.pallas import tpu as pltpu
from jax.experimental.pallas import tpu_sc as plsc

_THRESHOLD = 0.5
_NEGPOS_RATIO = 3
_VAR0 = 0.1
_VAR1 = 0.2

_P = 8732
_PPAD = 8736
_NSUB = _PPAD // 16
_T = 8
_C = 21
_B = 32
_ROWSTRIDE = _B * _PPAD
_BLOCKS = ((0, 2192), (2192, 2192), (4384, 2192), (6576, 2160))
_BUF = 2192

_LN2 = 0.6931471805599453
_SQRT2 = 1.4142135623730951


def _ln(x):
    bits = lax.bitcast_convert_type(x, jnp.int32)
    e = lax.shift_right_arithmetic(bits, 23) - 127
    mbits = lax.bitwise_or(lax.bitwise_and(bits, 0x007FFFFF), 0x3F800000)
    m = lax.bitcast_convert_type(mbits, jnp.float32)
    big = m > _SQRT2
    m = jnp.where(big, m * 0.5, m)
    ef = (e + jnp.where(big, 1, 0)).astype(jnp.float32)
    s = (m - 1.0) / (m + 1.0)
    s2 = s * s
    poly = 2.0 * s * (1.0 + s2 * (1.0 / 3.0 + s2 * (0.2 + s2 * (1.0 / 7.0))))
    return ef * _LN2 + poly


def _smooth_l1(d):
    ad = jnp.abs(d)
    return jnp.where(ad < 1.0, 0.5 * d * d, ad - 0.5)


def _splat_i(v):
    return jnp.full((16,), v, jnp.int32)


def _splat_f(v):
    return jnp.full((16,), v, jnp.float32)


def _sc_body(loc_hbm, conf_hbm, pri_hbm, tgt_hbm, out_hbm,
             pri_v, loc_v, conf_v, tgt_v, bto_v, bti_v, mine_v, res_v,
             rbuf_f, rbuf_i, sem):
    b = lax.axis_index("s") * 2 + lax.axis_index("c")
    iota = lax.iota(jnp.int32, 16)
    zf = _splat_f(0.0)
    zi = _splat_i(0)

    def _red(v, op, buf):
        for sh in (8, 4, 2, 1):
            buf[pl.ds(0, 16)] = v
            buf[pl.ds(16, 16)] = v
            v = op(v, buf[pl.ds(sh, 16)])
        return v

    def red_sum_f(v):
        return _red(v, jnp.add, rbuf_f)

    def red_max_f(v):
        return _red(v, jnp.maximum, rbuf_f)

    def red_sum_i(v):
        return _red(v, jnp.add, rbuf_i)

    def red_min_i(v):
        return _red(v, jnp.minimum, rbuf_i)

    pltpu.async_copy(tgt_hbm.at[pl.ds(b * (_T * 5 * 16), _T * 5 * 16)],
                     tgt_v, sem).wait()
    tb = [[tgt_v[pl.ds((t * 5 + c) * 16, 16)] for c in range(5)]
          for t in range(_T)]

    bpv = [_splat_f(-1.0) for _ in range(_T)]
    bpi = [zi for _ in range(_T)]

    for base, bs in _BLOCKS:
        cps = [pltpu.async_copy(
            pri_hbm.at[pl.ds(c * _PPAD + base, bs)],
            pri_v.at[pl.ds(c * _BUF, bs)], sem) for c in range(4)]
        for cp in cps:
            cp.wait()

        def pass_a(j, carry):
            bpv0 = list(carry[:_T])
            bpi0 = list(carry[_T:])
            off = j * 16
            gidx = iota + (base + off)
            valid = gidx < _P
            px = pri_v[pl.ds(off, 16)]
            py = pri_v[pl.ds(_BUF + off, 16)]
            pw = pri_v[pl.ds(2 * _BUF + off, 16)]
            ph = pri_v[pl.ds(3 * _BUF + off, 16)]
            p_x1 = px - pw * 0.5
            p_y1 = py - ph * 0.5
            p_x2 = px + pw * 0.5
            p_y2 = py + ph * 0.5
            area_b = (p_x2 - p_x1) * (p_y2 - p_y1)
            bto_c = zf
            bti_c = zi
            for t in range(_T):
                tx1, ty1, tx2, ty2, _ = tb[t]
                ix = jnp.maximum(
                    jnp.minimum(tx2, p_x2) - jnp.maximum(tx1, p_x1), 0.0)
                iy = jnp.maximum(
                    jnp.minimum(ty2, p_y2) - jnp.maximum(ty1, p_y1), 0.0)
                inter = ix * iy
                area_a = (tx2 - tx1) * (ty2 - ty1)
                ov = inter / (area_a + area_b - inter)
                ov = jnp.where(valid, ov, 0.0)
                upd = ov > bto_c
                bto_c = jnp.where(upd, ov, bto_c)
                bti_c = jnp.where(upd, _splat_i(t), bti_c)
                updp = ov > bpv0[t]
                bpv0[t] = jnp.where(updp, ov, bpv0[t])
                bpi0[t] = jnp.where(updp, gidx, bpi0[t])
            bto_v[pl.ds(base + off, 16)] = bto_c
            bti_v[pl.ds(base + off, 16)] = bti_c
            return tuple(bpv0) + tuple(bpi0)

        carry = lax.fori_loop(0, bs // 16, pass_a, tuple(bpv) + tuple(bpi))
        bpv = list(carry[:_T])
        bpi = list(carry[_T:])

    gmin = []
    for t in range(_T):
        mx = red_max_f(bpv[t])
        cand = bpv[t] == mx
        gmin.append(red_min_i(jnp.where(cand, bpi[t], _splat_i(_P))))

    acc_ll = zf
    acc_lc = zf
    acc_np = zi

    for base, bs in _BLOCKS:
        cps = [pltpu.async_copy(
            pri_hbm.at[pl.ds(c * _PPAD + base, bs)],
            pri_v.at[pl.ds(c * _BUF, bs)], sem) for c in range(4)]
        cps += [pltpu.async_copy(
            loc_hbm.at[pl.ds(c * _ROWSTRIDE + b * _PPAD + base, bs)],
            loc_v.at[pl.ds(c * _BUF, bs)], sem) for c in range(4)]
        cps += [pltpu.async_copy(
            conf_hbm.at[pl.ds(c * _ROWSTRIDE + b * _PPAD + base, bs)],
            conf_v.at[pl.ds(c * _BUF, bs)], sem) for c in range(_C)]
        for cp in cps:
            cp.wait()

        def pass_b(j, carry):
            a_ll, a_lc, a_np = carry
            off = j * 16
            gidx = iota + (base + off)
            valid = gidx < _P
            bto_c = bto_v[pl.ds(base + off, 16)]
            bti_c = bti_v[pl.ds(base + off, 16)]
            for t in range(_T):
                hit = gidx == gmin[t]
                bto_c = jnp.where(hit, 2.0, bto_c)
                bti_c = jnp.where(hit, _splat_i(t), bti_c)
            pos = jnp.logical_and(bto_c >= _THRESHOLD, valid)
            a_np = a_np + jnp.where(pos, 1, 0)

            m_x1, m_y1, m_x2, m_y2, m_lb = tb[0]
            for t in range(1, _T):
                sel = bti_c == t
                m_x1 = jnp.where(sel, tb[t][0], m_x1)
                m_y1 = jnp.where(sel, tb[t][1], m_y1)
                m_x2 = jnp.where(sel, tb[t][2], m_x2)
                m_y2 = jnp.where(sel, tb[t][3], m_y2)
                m_lb = jnp.where(sel, tb[t][4], m_lb)
            tgt_class = jnp.where(pos, m_lb.astype(jnp.int32) + 1, 0)

            px = pri_v[pl.ds(off, 16)]
            py = pri_v[pl.ds(_BUF + off, 16)]
            pw = pri_v[pl.ds(2 * _BUF + off, 16)]
            ph = pri_v[pl.ds(3 * _BUF + off, 16)]
            g_cx = ((m_x1 + m_x2) * 0.5 - px) / (_VAR0 * pw)
            g_cy = ((m_y1 + m_y2) * 0.5 - py) / (_VAR0 * ph)
            g_w = _ln((m_x2 - m_x1) / pw) * (1.0 / _VAR1)
            g_h = _ln((m_y2 - m_y1) / ph) * (1.0 / _VAR1)
            ll = (_smooth_l1(loc_v[pl.ds(off, 16)] - g_cx) +
                  _smooth_l1(loc_v[pl.ds(_BUF + off, 16)] - g_cy) +
                  _smooth_l1(loc_v[pl.ds(2 * _BUF + off, 16)] - g_w) +
                  _smooth_l1(loc_v[pl.ds(3 * _BUF + off, 16)] - g_h))
            a_ll = a_ll + jnp.where(pos, ll, 0.0)

            cv = [conf_v[pl.ds(c * _BUF + off, 16)] for c in range(_C)]
            cmax = cv[0]
            for c in range(1, _C):
                cmax = jnp.maximum(cmax, cv[c])
            ssum = zf
            conf_tgt = cv[0]
            for c in range(_C):
                ssum = ssum + jnp.exp(cv[c] - cmax)
                if c > 0:
                    conf_tgt = jnp.where(tgt_class == c, cv[c], conf_tgt)
            ce = _ln(ssum) + cmax - conf_tgt
            a_lc = a_lc + jnp.where(pos, ce, 0.0)
            mine = jnp.where(pos, 0.0, jnp.maximum(ce, 0.0))
            mine = jnp.where(valid, mine, 0.0)
            mine_v[pl.ds(base + off, 16)] = mine
            return a_ll, a_lc, a_np

        acc_ll, acc_lc, acc_np = lax.fori_loop(
            0, bs // 16, pass_b, (acc_ll, acc_lc, acc_np))

    num_pos_i = red_sum_i(acc_np)
    loss_l = red_sum_f(acc_ll)
    loss_c_pos = red_sum_f(acc_lc)
    k = jnp.minimum(_NEGPOS_RATIO * num_pos_i, _P - 1)

    def count_ge(midv):
        def cbody(j, acc):
            vi = lax.bitcast_convert_type(mine_v[pl.ds(j * 16, 16)],
                                          jnp.int32)
            return acc + jnp.where(vi >= midv, 1, 0)

        acc = lax.fori_loop(0, _NSUB, cbody, zi)
        return red_sum_i(acc)

    def bs_body(_, carry):
        lo, hi = carry
        mid = lo + lax.shift_right_arithmetic(hi - lo, 1)
        ge = count_ge(mid) >= k
        return jnp.where(ge, mid, lo), jnp.where(ge, hi, mid)

    lov, _hi = lax.fori_loop(0, 31, bs_body,
                             (zi, _splat_i(2139095041)))
    t_val = lax.bitcast_convert_type(lov, jnp.float32)

    def final_body(j, carry):
        a_cnt, a_sum = carry
        v = mine_v[pl.ds(j * 16, 16)]
        vi = lax.bitcast_convert_type(v, jnp.int32)
        gt = vi > lov
        return (a_cnt + jnp.where(gt, 1, 0),
                a_sum + jnp.where(gt, v, 0.0))

    cnt_gt, sum_gt = lax.fori_loop(0, _NSUB, final_body, (zi, zf))
    cnt_gt = red_sum_i(cnt_gt)
    sum_gt = red_sum_f(sum_gt)
    topk = sum_gt + (k - cnt_gt).astype(jnp.float32) * t_val
    loss_c = loss_c_pos + topk

    res = jnp.where(iota == 0, loss_l,
                    jnp.where(iota == 1, loss_c,
                              jnp.where(iota == 2,
                                        num_pos_i.astype(jnp.float32), zf)))
    res_v[...] = res
    pltpu.async_copy(res_v, out_hbm.at[pl.ds(b * 16, 16)], sem).wait()


@jax.jit
def kernel(loc_data, conf_data, priors, targets):
    pad = _PPAD - _P
    loc_f = jnp.pad(loc_data.transpose(2, 0, 1),
                    ((0, 0), (0, 0), (0, pad))).reshape(-1)
    conf_f = jnp.pad(conf_data.transpose(2, 0, 1),
                     ((0, 0), (0, 0), (0, pad))).reshape(-1)
    pri_f = jnp.pad(priors.T, ((0, 0), (0, pad))).reshape(-1)
    tgt_f = jnp.broadcast_to(
        targets.reshape(_B, _T * 5, 1), (_B, _T * 5, 16)).reshape(-1)

    mesh = plsc.VectorSubcoreMesh(core_axis_name="c", subcore_axis_name="s")
    run = functools.partial(
        pl.kernel,
        mesh=mesh,
        out_type=jax.ShapeDtypeStruct((_B * 16,), jnp.float32),
        scratch_types=[
            pltpu.VMEM((4 * _BUF,), jnp.float32),
            pltpu.VMEM((4 * _BUF,), jnp.float32),
            pltpu.VMEM((_C * _BUF,), jnp.float32),
            pltpu.VMEM((_T * 5 * 16,), jnp.float32),
            pltpu.VMEM((_PPAD,), jnp.float32),
            pltpu.VMEM((_PPAD,), jnp.int32),
            pltpu.VMEM((_PPAD,), jnp.float32),
            pltpu.VMEM((16,), jnp.float32),
            pltpu.VMEM((32,), jnp.float32),
            pltpu.VMEM((32,), jnp.int32),
            pltpu.SemaphoreType.DMA,
        ],
    )(_sc_body)
    out = run(loc_f, conf_f, pri_f, tgt_f).reshape(_B, 16)
    ll = jnp.sum(out[:, 0])
    lc = jnp.sum(out[:, 1])
    n = jnp.sum(out[:, 2])
    return (ll / n, lc / n)


# speedup vs baseline: 1.2501x; 1.2501x over previous
import functools

import jax
import jax.numpy as jnp
from jax import lax
from jax.experimental import pallas as pl
from jax.experimental.pallas import tpu as pltpu
from jax.experimental.pallas import tpu_sc as plsc

_THRESHOLD = 0.5
_NEGPOS_RATIO = 3
_VAR0 = 0.1
_VAR1 = 0.2

_P = 8732
_PPAD = 8736
_NSUB = _PPAD // 16
_T = 8
_C = 21
_B = 32
_ROWSTRIDE = _B * _PPAD
_BLOCKS = ((0, 2192), (2192, 2192), (4384, 2192), (6576, 2160))
_BUF = 2192
_UNROLL = 13

_LN2 = 0.6931471805599453
_SQRT2 = 1.4142135623730951


def _ln(x):
    bits = lax.bitcast_convert_type(x, jnp.int32)
    e = lax.shift_right_arithmetic(bits, 23) - 127
    mbits = lax.bitwise_or(lax.bitwise_and(bits, 0x007FFFFF), 0x3F800000)
    m = lax.bitcast_convert_type(mbits, jnp.float32)
    big = m > _SQRT2
    m = jnp.where(big, m * 0.5, m)
    ef = (e + jnp.where(big, 1, 0)).astype(jnp.float32)
    s = (m - 1.0) / (m + 1.0)
    s2 = s * s
    poly = 2.0 * s * (1.0 + s2 * (1.0 / 3.0 + s2 * (0.2 + s2 * (1.0 / 7.0))))
    return ef * _LN2 + poly


def _smooth_l1(d):
    ad = jnp.abs(d)
    return jnp.where(ad < 1.0, 0.5 * d * d, ad - 0.5)


def _splat_i(v):
    return jnp.full((16,), v, jnp.int32)


def _splat_f(v):
    return jnp.full((16,), v, jnp.float32)


def _sc_body(loc_hbm, conf_hbm, pri_hbm, tgt_hbm, out_hbm,
             pri_v, loc_v, conf_v, tgt_v, bto_v, bti_v, mine_v, res_v,
             rbuf_f, rbuf_i, sem):
    b = lax.axis_index("s") * 2 + lax.axis_index("c")
    iota = lax.iota(jnp.int32, 16)
    zf = _splat_f(0.0)
    zi = _splat_i(0)

    def _red(v, op, buf):
        for sh in (8, 4, 2, 1):
            buf[pl.ds(0, 16)] = v
            buf[pl.ds(16, 16)] = v
            v = op(v, buf[pl.ds(sh, 16)])
        return v

    def red_sum_f(v):
        return _red(v, jnp.add, rbuf_f)

    def red_max_f(v):
        return _red(v, jnp.maximum, rbuf_f)

    def red_sum_i(v):
        return _red(v, jnp.add, rbuf_i)

    def red_min_i(v):
        return _red(v, jnp.minimum, rbuf_i)

    pltpu.async_copy(tgt_hbm.at[pl.ds(b * (_T * 5 * 16), _T * 5 * 16)],
                     tgt_v, sem).wait()
    tb = [[tgt_v[pl.ds((t * 5 + c) * 16, 16)] for c in range(5)]
          for t in range(_T)]

    bpv = [_splat_f(-1.0) for _ in range(_T)]
    bpi = [zi for _ in range(_T)]

    for base, bs in _BLOCKS:
        cps = [pltpu.async_copy(
            pri_hbm.at[pl.ds(c * _PPAD + base, bs)],
            pri_v.at[pl.ds(c * _BUF, bs)], sem) for c in range(4)]
        for cp in cps:
            cp.wait()

        def pass_a(j, carry):
            bpv0 = list(carry[:_T])
            bpi0 = list(carry[_T:])
            off = j * 16
            gidx = iota + (base + off)
            valid = gidx < _P
            px = pri_v[pl.ds(off, 16)]
            py = pri_v[pl.ds(_BUF + off, 16)]
            pw = pri_v[pl.ds(2 * _BUF + off, 16)]
            ph = pri_v[pl.ds(3 * _BUF + off, 16)]
            p_x1 = px - pw * 0.5
            p_y1 = py - ph * 0.5
            p_x2 = px + pw * 0.5
            p_y2 = py + ph * 0.5
            area_b = (p_x2 - p_x1) * (p_y2 - p_y1)
            bto_c = zf
            bti_c = zi
            for t in range(_T):
                tx1, ty1, tx2, ty2, _ = tb[t]
                ix = jnp.maximum(
                    jnp.minimum(tx2, p_x2) - jnp.maximum(tx1, p_x1), 0.0)
                iy = jnp.maximum(
                    jnp.minimum(ty2, p_y2) - jnp.maximum(ty1, p_y1), 0.0)
                inter = ix * iy
                area_a = (tx2 - tx1) * (ty2 - ty1)
                ov = inter / (area_a + area_b - inter)
                ov = jnp.where(valid, ov, 0.0)
                upd = ov > bto_c
                bto_c = jnp.where(upd, ov, bto_c)
                bti_c = jnp.where(upd, _splat_i(t), bti_c)
                updp = ov > bpv0[t]
                bpv0[t] = jnp.where(updp, ov, bpv0[t])
                bpi0[t] = jnp.where(updp, gidx, bpi0[t])
            bto_v[pl.ds(base + off, 16)] = bto_c
            bti_v[pl.ds(base + off, 16)] = bti_c
            return tuple(bpv0) + tuple(bpi0)

        carry = lax.fori_loop(0, bs // 16, pass_a, tuple(bpv) + tuple(bpi))
        bpv = list(carry[:_T])
        bpi = list(carry[_T:])

    gmin = []
    for t in range(_T):
        mx = red_max_f(bpv[t])
        cand = bpv[t] == mx
        gmin.append(red_min_i(jnp.where(cand, bpi[t], _splat_i(_P))))

    acc_ll = zf
    acc_lc = zf
    acc_np = zi

    for base, bs in _BLOCKS:
        cps = [pltpu.async_copy(
            pri_hbm.at[pl.ds(c * _PPAD + base, bs)],
            pri_v.at[pl.ds(c * _BUF, bs)], sem) for c in range(4)]
        cps += [pltpu.async_copy(
            loc_hbm.at[pl.ds(c * _ROWSTRIDE + b * _PPAD + base, bs)],
            loc_v.at[pl.ds(c * _BUF, bs)], sem) for c in range(4)]
        cps += [pltpu.async_copy(
            conf_hbm.at[pl.ds(c * _ROWSTRIDE + b * _PPAD + base, bs)],
            conf_v.at[pl.ds(c * _BUF, bs)], sem) for c in range(_C)]
        for cp in cps:
            cp.wait()

        def pass_b(j, carry):
            a_ll, a_lc, a_np = carry
            off = j * 16
            gidx = iota + (base + off)
            valid = gidx < _P
            bto_c = bto_v[pl.ds(base + off, 16)]
            bti_c = bti_v[pl.ds(base + off, 16)]
            for t in range(_T):
                hit = gidx == gmin[t]
                bto_c = jnp.where(hit, 2.0, bto_c)
                bti_c = jnp.where(hit, _splat_i(t), bti_c)
            pos = jnp.logical_and(bto_c >= _THRESHOLD, valid)
            a_np = a_np + jnp.where(pos, 1, 0)

            m_x1, m_y1, m_x2, m_y2, m_lb = tb[0]
            for t in range(1, _T):
                sel = bti_c == t
                m_x1 = jnp.where(sel, tb[t][0], m_x1)
                m_y1 = jnp.where(sel, tb[t][1], m_y1)
                m_x2 = jnp.where(sel, tb[t][2], m_x2)
                m_y2 = jnp.where(sel, tb[t][3], m_y2)
                m_lb = jnp.where(sel, tb[t][4], m_lb)
            tgt_class = jnp.where(pos, m_lb.astype(jnp.int32) + 1, 0)

            px = pri_v[pl.ds(off, 16)]
            py = pri_v[pl.ds(_BUF + off, 16)]
            pw = pri_v[pl.ds(2 * _BUF + off, 16)]
            ph = pri_v[pl.ds(3 * _BUF + off, 16)]
            g_cx = ((m_x1 + m_x2) * 0.5 - px) / (_VAR0 * pw)
            g_cy = ((m_y1 + m_y2) * 0.5 - py) / (_VAR0 * ph)
            g_w = _ln((m_x2 - m_x1) / pw) * (1.0 / _VAR1)
            g_h = _ln((m_y2 - m_y1) / ph) * (1.0 / _VAR1)
            ll = (_smooth_l1(loc_v[pl.ds(off, 16)] - g_cx) +
                  _smooth_l1(loc_v[pl.ds(_BUF + off, 16)] - g_cy) +
                  _smooth_l1(loc_v[pl.ds(2 * _BUF + off, 16)] - g_w) +
                  _smooth_l1(loc_v[pl.ds(3 * _BUF + off, 16)] - g_h))
            a_ll = a_ll + jnp.where(pos, ll, 0.0)

            cv = [conf_v[pl.ds(c * _BUF + off, 16)] for c in range(_C)]
            cmax = cv[0]
            for c in range(1, _C):
                cmax = jnp.maximum(cmax, cv[c])
            ssum = zf
            conf_tgt = cv[0]
            for c in range(_C):
                ssum = ssum + jnp.exp(cv[c] - cmax)
                if c > 0:
                    conf_tgt = jnp.where(tgt_class == c, cv[c], conf_tgt)
            ce = _ln(ssum) + cmax - conf_tgt
            a_lc = a_lc + jnp.where(pos, ce, 0.0)
            mine = jnp.where(pos, 0.0, jnp.maximum(ce, 0.0))
            mine = jnp.where(valid, mine, 0.0)
            mine_v[pl.ds(base + off, 16)] = mine
            return a_ll, a_lc, a_np

        acc_ll, acc_lc, acc_np = lax.fori_loop(
            0, bs // 16, pass_b, (acc_ll, acc_lc, acc_np))

    num_pos_i = red_sum_i(acc_np)
    loss_l = red_sum_f(acc_ll)
    loss_c_pos = red_sum_f(acc_lc)
    k = jnp.minimum(_NEGPOS_RATIO * num_pos_i, _P - 1)

    def count_ge(midv):
        def cbody(j, acc):
            base = j * (16 * _UNROLL)
            for u in range(_UNROLL):
                vi = lax.bitcast_convert_type(
                    mine_v[pl.ds(base + u * 16, 16)], jnp.int32)
                acc = acc + jnp.where(vi >= midv, 1, 0)
            return acc

        acc = lax.fori_loop(0, _NSUB // _UNROLL, cbody, zi)
        return red_sum_i(acc)

    def bs_body(_, carry):
        lo, hi = carry
        mid = lo + lax.shift_right_arithmetic(hi - lo, 1)
        ge = count_ge(mid) >= k
        return jnp.where(ge, mid, lo), jnp.where(ge, hi, mid)

    lov, _hi = lax.fori_loop(0, 31, bs_body,
                             (zi, _splat_i(2139095041)))
    t_val = lax.bitcast_convert_type(lov, jnp.float32)

    def final_body(j, carry):
        a_cnt, a_sum = carry
        base = j * (16 * _UNROLL)
        for u in range(_UNROLL):
            v = mine_v[pl.ds(base + u * 16, 16)]
            vi = lax.bitcast_convert_type(v, jnp.int32)
            gt = vi > lov
            a_cnt = a_cnt + jnp.where(gt, 1, 0)
            a_sum = a_sum + jnp.where(gt, v, 0.0)
        return a_cnt, a_sum

    cnt_gt, sum_gt = lax.fori_loop(0, _NSUB // _UNROLL, final_body, (zi, zf))
    cnt_gt = red_sum_i(cnt_gt)
    sum_gt = red_sum_f(sum_gt)
    topk = sum_gt + (k - cnt_gt).astype(jnp.float32) * t_val
    loss_c = loss_c_pos + topk

    res = jnp.where(iota == 0, loss_l,
                    jnp.where(iota == 1, loss_c,
                              jnp.where(iota == 2,
                                        num_pos_i.astype(jnp.float32), zf)))
    res_v[...] = res
    pltpu.async_copy(res_v, out_hbm.at[pl.ds(b * 16, 16)], sem).wait()


@jax.jit
def kernel(loc_data, conf_data, priors, targets):
    pad = _PPAD - _P
    loc_f = jnp.pad(loc_data, ((0, 0), (0, pad), (0, 0))
                    ).transpose(2, 0, 1).reshape(-1)
    conf_f = jnp.pad(conf_data, ((0, 0), (0, pad), (0, 0))
                     ).transpose(2, 0, 1).reshape(-1)
    pri_f = jnp.pad(priors.T, ((0, 0), (0, pad))).reshape(-1)
    tgt_f = jnp.broadcast_to(
        targets.reshape(_B, _T * 5, 1), (_B, _T * 5, 16)).reshape(-1)

    mesh = plsc.VectorSubcoreMesh(core_axis_name="c", subcore_axis_name="s")
    run = functools.partial(
        pl.kernel,
        mesh=mesh,
        out_type=jax.ShapeDtypeStruct((_B * 16,), jnp.float32),
        scratch_types=[
            pltpu.VMEM((4 * _BUF,), jnp.float32),
            pltpu.VMEM((4 * _BUF,), jnp.float32),
            pltpu.VMEM((_C * _BUF,), jnp.float32),
            pltpu.VMEM((_T * 5 * 16,), jnp.float32),
            pltpu.VMEM((_PPAD,), jnp.float32),
            pltpu.VMEM((_PPAD,), jnp.int32),
            pltpu.VMEM((_PPAD,), jnp.float32),
            pltpu.VMEM((16,), jnp.float32),
            pltpu.VMEM((32,), jnp.float32),
            pltpu.VMEM((32,), jnp.int32),
            pltpu.SemaphoreType.DMA,
        ],
    )(_sc_body)
    out = run(loc_f, conf_f, pri_f, tgt_f).reshape(_B, 16)
    ll = jnp.sum(out[:, 0])
    lc = jnp.sum(out[:, 1])
    n = jnp.sum(out[:, 2])
    return (ll / n, lc / n)
